# baseline (speedup 1.0000x reference)
import jax
import jax.numpy as jnp
from jax import lax
from jax.experimental import pallas as pl
from jax.experimental.pallas import tpu as pltpu
from jax.experimental.pallas import tpu_sc as plsc

BS = 4
N = 10000
E = 160000
D = 128
T = 16

NC = 2
NS = 16
L = 16

CH = 64
SUP = 16
DEPTH = 5
PER_TILE = 10240
NCHUNK = PER_TILE // CH
NSUP = NCHUNK // SUP
ROWS_PER_TILE = 624
TAIL_ROWS = N - NS * ROWS_PER_TILE
ACC_ROWS = N + 16

CHH = 128
SUPH = 8
NCHUNK_H = PER_TILE // CHH
NSUP_H = NCHUNK_H // SUPH
HROWS_PER_TILE = 640
DUMP = NS * HROWS_PER_TILE
HIST_ROWS = DUMP + 16
PACKED_PER_TILE = HROWS_PER_TILE // 8
PACKED = NS * PACKED_PER_TILE


def _mm_body(x_ref, w_ref, o_ref):
    o_ref[...] = lax.dot_general(
        x_ref[...], w_ref[...], (((1,), (1,)), ((), ())),
        preferred_element_type=jnp.float32)


def _mid_body(x_ref, r_ref, cnt_ref, tbl_ref, w_ref, e_ref, t_ref):
    z = r_ref[...] + lax.dot_general(
        cnt_ref[...], tbl_ref[...], (((1,), (0,)), ((), ())),
        preferred_element_type=jnp.float32)
    e = x_ref[...] + jax.nn.sigmoid(z)
    e_ref[...] = e
    t_ref[...] = lax.dot_general(
        e, w_ref[...], (((1,), (1,)), ((), ())),
        preferred_element_type=jnp.float32)


def _post_body(x_ref, r_ref, cnt_ref, tbl_ref, o_ref):
    z = r_ref[...] + lax.dot_general(
        cnt_ref[...], tbl_ref[...], (((1,), (0,)), ((), ())),
        preferred_element_type=jnp.float32)
    o_ref[...] = x_ref[...] + jax.nn.sigmoid(z)


def _tbl_body(emb_ref, w_ref, b_ref, o_ref):
    o_ref[...] = lax.dot_general(
        emb_ref[...], w_ref[...], (((1,), (1,)), ((), ())),
        preferred_element_type=jnp.float32) + b_ref[...]


_BLK = 2000
_GRID = (BS * N) // _BLK


def _transform(x_flat, wg):
    return pl.pallas_call(
        _mm_body,
        grid=(_GRID,),
        in_specs=[pl.BlockSpec((_BLK, D), lambda i: (i, 0)),
                  pl.BlockSpec((D, D), lambda i: (0, 0))],
        out_specs=pl.BlockSpec((_BLK, D), lambda i: (i, 0)),
        out_shape=jax.ShapeDtypeStruct((BS * N, D), jnp.float32),
    )(x_flat, wg)


def _mid(x_flat, res_flat, cnt_flat, tbl, wg):
    return pl.pallas_call(
        _mid_body,
        grid=(_GRID,),
        in_specs=[pl.BlockSpec((_BLK, D), lambda i: (i, 0)),
                  pl.BlockSpec((_BLK, D), lambda i: (i, 0)),
                  pl.BlockSpec((_BLK, T), lambda i: (i, 0)),
                  pl.BlockSpec((T, D), lambda i: (0, 0)),
                  pl.BlockSpec((D, D), lambda i: (0, 0))],
        out_specs=[pl.BlockSpec((_BLK, D), lambda i: (i, 0)),
                   pl.BlockSpec((_BLK, D), lambda i: (i, 0))],
        out_shape=[jax.ShapeDtypeStruct((BS * N, D), jnp.float32),
                   jax.ShapeDtypeStruct((BS * N, D), jnp.float32)],
    )(x_flat, res_flat, cnt_flat, tbl, wg)


def _post(x_flat, res_flat, cnt_flat, tbl):
    return pl.pallas_call(
        _post_body,
        grid=(_GRID,),
        in_specs=[pl.BlockSpec((_BLK, D), lambda i: (i, 0)),
                  pl.BlockSpec((_BLK, D), lambda i: (i, 0)),
                  pl.BlockSpec((_BLK, T), lambda i: (i, 0)),
                  pl.BlockSpec((T, D), lambda i: (0, 0))],
        out_specs=pl.BlockSpec((_BLK, D), lambda i: (i, 0)),
        out_shape=jax.ShapeDtypeStruct((BS * N, D), jnp.float32),
    )(x_flat, res_flat, cnt_flat, tbl)


def _type_table(emb_table, we, b):
    return pl.pallas_call(
        _tbl_body,
        in_specs=[pl.BlockSpec((T, D), lambda: (0, 0)),
                  pl.BlockSpec((D, D), lambda: (0, 0)),
                  pl.BlockSpec((1, D), lambda: (0, 0))],
        out_specs=pl.BlockSpec((T, D), lambda: (0, 0)),
        out_shape=jax.ShapeDtypeStruct((T, D), jnp.float32),
    )(emb_table, we, b.reshape(1, D))


FLAT = NS * HROWS_PER_TILE * T
HDUMP = FLAT
HIST_WORDS = FLAT + 256


def _hist_body(hidx_hbm, zerosf_hbm, hist_hbm, hv, ones, histo):
    c = lax.axis_index("c")
    s = lax.axis_index("s")
    wpt = HROWS_PER_TILE * T
    for k in range(CHH // L):
        ones[pl.ds(k * L, L)] = jnp.full((L,), 1.0, jnp.float32)
    for p in range(2):
        b = c * 2 + p
        pltpu.sync_copy(zerosf_hbm, histo.at[pl.ds(s * wpt, wpt)])
        plsc.subcore_barrier()

        def super_chunk(u, carry):
            pltpu.sync_copy(hidx_hbm.at[b, s, pl.ds(u * SUPH, SUPH)], hv)
            for jj in range(SUPH):
                pltpu.sync_copy(ones, histo.at[hv.at[jj]], add=True)
            return carry

        lax.fori_loop(0, NSUP_H, super_chunk, 0)
        plsc.subcore_barrier()
        pltpu.sync_copy(histo.at[pl.ds(s * wpt, wpt)],
                        hist_hbm.at[b, pl.ds(s * wpt, wpt)])
        plsc.subcore_barrier()


def _sc_hist(hidxp, zerosf):
    mesh = plsc.VectorSubcoreMesh(core_axis_name="c", subcore_axis_name="s")
    return pl.kernel(
        _hist_body,
        out_type=jax.ShapeDtypeStruct((BS, FLAT), jnp.float32),
        mesh=mesh,
        scratch_types=[
            pltpu.VMEM((SUPH, CHH), jnp.int32),
            pltpu.VMEM((CHH,), jnp.float32),
            pltpu.VMEM_SHARED((HIST_WORDS,), jnp.float32),
        ],
    )(hidxp, zerosf)


def _sc_body(ext_hbm, gidx_hbm, sidx_hbm, zeros_hbm, out_hbm,
             gv, sv, rows0, rows1, rows2, rows3, rows4, acc,
             sem0, sem1, sem2, sem3, sem4):
    c = lax.axis_index("c")
    s = lax.axis_index("s")
    rows = (rows0, rows1, rows2, rows3, rows4)
    sems = (sem0, sem1, sem2, sem3, sem4)
    for p in range(2):
        b = c * 2 + p
        pltpu.sync_copy(zeros_hbm, acc.at[pl.ds(s * ROWS_PER_TILE, ROWS_PER_TILE)])

        @pl.when(s == NS - 1)
        def _zero_tail():
            pltpu.sync_copy(zeros_hbm.at[pl.ds(0, TAIL_ROWS)],
                            acc.at[pl.ds(NS * ROWS_PER_TILE, TAIL_ROWS)])

        plsc.subcore_barrier()

        def super_chunk(u, carry):
            pltpu.sync_copy(gidx_hbm.at[b, s, pl.ds(u * SUP, SUP)], gv)
            pltpu.sync_copy(sidx_hbm.at[b, s, pl.ds(u * SUP, SUP)], sv)
            copies = [None] * DEPTH
            for k in range(DEPTH):
                copies[k] = pltpu.async_copy(
                    ext_hbm.at[gv.at[k]], rows[k], sems[k])
            for jj in range(SUP):
                bb = jj % DEPTH
                copies[bb].wait()
                pltpu.sync_copy(rows[bb], acc.at[sv.at[jj]], add=True)
                if jj + DEPTH < SUP:
                    copies[bb] = pltpu.async_copy(
                        ext_hbm.at[gv.at[jj + DEPTH]], rows[bb], sems[bb])
            return carry

        lax.fori_loop(0, NSUP, super_chunk, 0)
        plsc.subcore_barrier()
        pltpu.sync_copy(acc.at[pl.ds(s * ROWS_PER_TILE, ROWS_PER_TILE)],
                        out_hbm.at[b, pl.ds(s * ROWS_PER_TILE, ROWS_PER_TILE)])

        @pl.when(s == NS - 1)
        def _copy_tail():
            pltpu.sync_copy(acc.at[pl.ds(NS * ROWS_PER_TILE, TAIL_ROWS)],
                            out_hbm.at[b, pl.ds(NS * ROWS_PER_TILE, TAIL_ROWS)])

        plsc.subcore_barrier()


def _sc_scatter(ext, gidx, sidx, zeros_rows):
    mesh = plsc.VectorSubcoreMesh(core_axis_name="c", subcore_axis_name="s")
    return pl.kernel(
        _sc_body,
        out_type=jax.ShapeDtypeStruct((BS, N, D), jnp.float32),
        mesh=mesh,
        scratch_types=[
            pltpu.VMEM((SUP, CH), jnp.int32),
            pltpu.VMEM((SUP, CH), jnp.int32),
            pltpu.VMEM((CH, D), jnp.float32),
            pltpu.VMEM((CH, D), jnp.float32),
            pltpu.VMEM((CH, D), jnp.float32),
            pltpu.VMEM((CH, D), jnp.float32),
            pltpu.VMEM((CH, D), jnp.float32),
            pltpu.VMEM_SHARED((ACC_ROWS, D), jnp.float32),
            pltpu.SemaphoreType.DMA,
            pltpu.SemaphoreType.DMA,
            pltpu.SemaphoreType.DMA,
            pltpu.SemaphoreType.DMA,
            pltpu.SemaphoreType.DMA,
        ],
    )(ext, gidx, sidx, zeros_rows)


def kernel(node_embedding, edges, edge_types, emb_table, W0, b0, W1, b1):
    x = node_embedding.reshape(BS * N, D)

    origin = edges[:, :, 0]
    final = edges[:, :, 1]
    npad = NS * PER_TILE - E

    goff = origin + (jnp.arange(BS, dtype=jnp.int32) * N)[:, None]
    pad_g = jnp.arange(npad, dtype=jnp.int32) % 4096
    pad_g = jnp.broadcast_to(pad_g, (BS, npad))
    pad_s = N + (jnp.arange(npad, dtype=jnp.int32) % 16)
    pad_s = jnp.broadcast_to(pad_s, (BS, npad))
    gidx = jnp.concatenate([goff, pad_g], axis=1).reshape(BS, NS, NCHUNK, CH)
    sidx = jnp.concatenate([final, pad_s], axis=1).reshape(BS, NS, NCHUNK, CH)

    pad_h = HDUMP + (jnp.arange(npad, dtype=jnp.int32) % 256)
    pad_h = jnp.broadcast_to(pad_h, (BS, npad))
    hidxp = jnp.concatenate([final * T + edge_types, pad_h], axis=1).reshape(
        BS, NS, NCHUNK_H, CHH)

    zeros_rows = jnp.zeros((ROWS_PER_TILE, D), jnp.float32)
    zerosf = jnp.zeros((HROWS_PER_TILE * T,), jnp.float32)

    hist = _sc_hist(hidxp, zerosf)
    cnt_flat = hist.reshape(BS, NS * HROWS_PER_TILE, T)[:, :N, :].reshape(
        BS * N, T)

    cur = x
    res = None
    tbls = []
    for li, (W, b) in enumerate(((W0, b0), (W1, b1))):
        wg = W[:, :D]
        tbls.append(_type_table(emb_table, W[:, D:], b))
        if li == 0:
            transformed = _transform(cur, wg)
        else:
            cur, transformed = _mid(cur, res, cnt_flat, tbls[0], wg)
        res = _sc_scatter(transformed, gidx, sidx, zeros_rows).reshape(BS * N, D)

    out = _post(cur, res, cnt_flat, tbls[1])
    return out.reshape(BS, N, D)


# speedup vs baseline: 1.0140x; 1.0140x over previous
import jax
import jax.numpy as jnp
from jax import lax
from jax.experimental import pallas as pl
from jax.experimental.pallas import tpu as pltpu
from jax.experimental.pallas import tpu_sc as plsc

BS = 4
N = 10000
E = 160000
D = 128
T = 16

NC = 2
NS = 16
L = 16

CH = 64
SUP = 16
DEPTH = 4
PER_TILE = 10240
NCHUNK = PER_TILE // CH
NSUP = NCHUNK // SUP
ROWS_PER_TILE = 624
TAIL_ROWS = N - NS * ROWS_PER_TILE
ACC_ROWS = N + 16

CHH = 128
SUPH = 8
NCHUNK_H = PER_TILE // CHH
NSUP_H = NCHUNK_H // SUPH
HROWS_PER_TILE = 640
DUMP = NS * HROWS_PER_TILE
HIST_ROWS = DUMP + 16
PACKED_PER_TILE = HROWS_PER_TILE // 8
PACKED = NS * PACKED_PER_TILE


def _mm_body(x_ref, w_ref, o_ref):
    o_ref[...] = lax.dot_general(
        x_ref[...], w_ref[...], (((1,), (1,)), ((), ())),
        preferred_element_type=jnp.float32)


def _mid_body(x_ref, r_ref, cnt_ref, tbl_ref, w_ref, e_ref, t_ref):
    z = r_ref[...] + lax.dot_general(
        cnt_ref[...], tbl_ref[...], (((1,), (0,)), ((), ())),
        preferred_element_type=jnp.float32)
    e = x_ref[...] + jax.nn.sigmoid(z)
    e_ref[...] = e
    t_ref[...] = lax.dot_general(
        e, w_ref[...], (((1,), (1,)), ((), ())),
        preferred_element_type=jnp.float32)


def _post_body(x_ref, r_ref, cnt_ref, tbl_ref, o_ref):
    z = r_ref[...] + lax.dot_general(
        cnt_ref[...], tbl_ref[...], (((1,), (0,)), ((), ())),
        preferred_element_type=jnp.float32)
    o_ref[...] = x_ref[...] + jax.nn.sigmoid(z)


def _tbl_body(emb_ref, w_ref, b_ref, o_ref):
    o_ref[...] = lax.dot_general(
        emb_ref[...], w_ref[...], (((1,), (1,)), ((), ())),
        preferred_element_type=jnp.float32) + b_ref[...]


_BLK = 2000
_GRID = (BS * N) // _BLK


def _transform(x_flat, wg):
    return pl.pallas_call(
        _mm_body,
        grid=(_GRID,),
        in_specs=[pl.BlockSpec((_BLK, D), lambda i: (i, 0)),
                  pl.BlockSpec((D, D), lambda i: (0, 0))],
        out_specs=pl.BlockSpec((_BLK, D), lambda i: (i, 0)),
        out_shape=jax.ShapeDtypeStruct((BS * N, D), jnp.float32),
    )(x_flat, wg)


def _mid(x_flat, res_flat, cnt_flat, tbl, wg):
    return pl.pallas_call(
        _mid_body,
        grid=(_GRID,),
        in_specs=[pl.BlockSpec((_BLK, D), lambda i: (i, 0)),
                  pl.BlockSpec((_BLK, D), lambda i: (i, 0)),
                  pl.BlockSpec((_BLK, T), lambda i: (i, 0)),
                  pl.BlockSpec((T, D), lambda i: (0, 0)),
                  pl.BlockSpec((D, D), lambda i: (0, 0))],
        out_specs=[pl.BlockSpec((_BLK, D), lambda i: (i, 0)),
                   pl.BlockSpec((_BLK, D), lambda i: (i, 0))],
        out_shape=[jax.ShapeDtypeStruct((BS * N, D), jnp.float32),
                   jax.ShapeDtypeStruct((BS * N, D), jnp.float32)],
    )(x_flat, res_flat, cnt_flat, tbl, wg)


def _post(x_flat, res_flat, cnt_flat, tbl):
    return pl.pallas_call(
        _post_body,
        grid=(_GRID,),
        in_specs=[pl.BlockSpec((_BLK, D), lambda i: (i, 0)),
                  pl.BlockSpec((_BLK, D), lambda i: (i, 0)),
                  pl.BlockSpec((_BLK, T), lambda i: (i, 0)),
                  pl.BlockSpec((T, D), lambda i: (0, 0))],
        out_specs=pl.BlockSpec((_BLK, D), lambda i: (i, 0)),
        out_shape=jax.ShapeDtypeStruct((BS * N, D), jnp.float32),
    )(x_flat, res_flat, cnt_flat, tbl)


def _type_table(emb_table, we, b):
    return pl.pallas_call(
        _tbl_body,
        in_specs=[pl.BlockSpec((T, D), lambda: (0, 0)),
                  pl.BlockSpec((D, D), lambda: (0, 0)),
                  pl.BlockSpec((1, D), lambda: (0, 0))],
        out_specs=pl.BlockSpec((T, D), lambda: (0, 0)),
        out_shape=jax.ShapeDtypeStruct((T, D), jnp.float32),
    )(emb_table, we, b.reshape(1, D))


FLAT = NS * HROWS_PER_TILE * T
HDUMP = FLAT
HIST_WORDS = FLAT + 256


def _hist_body(hidx_hbm, zerosf_hbm, hist_hbm, hv, ones, histo):
    c = lax.axis_index("c")
    s = lax.axis_index("s")
    wpt = HROWS_PER_TILE * T
    for k in range(CHH // L):
        ones[pl.ds(k * L, L)] = jnp.full((L,), 1.0, jnp.float32)
    for p in range(2):
        b = c * 2 + p
        pltpu.sync_copy(zerosf_hbm, histo.at[pl.ds(s * wpt, wpt)])
        plsc.subcore_barrier()

        def super_chunk(u, carry):
            pltpu.sync_copy(hidx_hbm.at[b, s, pl.ds(u * SUPH, SUPH)], hv)
            for jj in range(SUPH):
                pltpu.sync_copy(ones, histo.at[hv.at[jj]], add=True)
            return carry

        lax.fori_loop(0, NSUP_H, super_chunk, 0)
        plsc.subcore_barrier()
        pltpu.sync_copy(histo.at[pl.ds(s * wpt, wpt)],
                        hist_hbm.at[b, pl.ds(s * wpt, wpt)])
        plsc.subcore_barrier()


def _sc_hist(hidxp, zerosf):
    mesh = plsc.VectorSubcoreMesh(core_axis_name="c", subcore_axis_name="s")
    return pl.kernel(
        _hist_body,
        out_type=jax.ShapeDtypeStruct((BS, FLAT), jnp.float32),
        mesh=mesh,
        scratch_types=[
            pltpu.VMEM((SUPH, CHH), jnp.int32),
            pltpu.VMEM((CHH,), jnp.float32),
            pltpu.VMEM_SHARED((HIST_WORDS,), jnp.float32),
        ],
    )(hidxp, zerosf)


def _sc_body(ext_hbm, gidx_hbm, sidx_hbm, zeros_hbm, out_hbm,
             gv, sv, rows0, rows1, rows2, rows3, acc,
             sem0, sem1, sem2, sem3):
    c = lax.axis_index("c")
    s = lax.axis_index("s")
    rows = (rows0, rows1, rows2, rows3)
    sems = (sem0, sem1, sem2, sem3)
    for p in range(2):
        b = c * 2 + p
        pltpu.sync_copy(zeros_hbm, acc.at[pl.ds(s * ROWS_PER_TILE, ROWS_PER_TILE)])

        @pl.when(s == NS - 1)
        def _zero_tail():
            pltpu.sync_copy(zeros_hbm.at[pl.ds(0, TAIL_ROWS)],
                            acc.at[pl.ds(NS * ROWS_PER_TILE, TAIL_ROWS)])

        plsc.subcore_barrier()

        def super_chunk(u, carry):
            pltpu.sync_copy(gidx_hbm.at[b, s, pl.ds(u * SUP, SUP)], gv)
            pltpu.sync_copy(sidx_hbm.at[b, s, pl.ds(u * SUP, SUP)], sv)
            copies = [None] * DEPTH
            for k in range(DEPTH):
                copies[k] = pltpu.async_copy(
                    ext_hbm.at[gv.at[k]], rows[k], sems[k])
            for jj in range(SUP):
                bb = jj % DEPTH
                copies[bb].wait()
                pltpu.sync_copy(rows[bb], acc.at[sv.at[jj]], add=True)
                if jj + DEPTH < SUP:
                    copies[bb] = pltpu.async_copy(
                        ext_hbm.at[gv.at[jj + DEPTH]], rows[bb], sems[bb])
            return carry

        lax.fori_loop(0, NSUP, super_chunk, 0)
        plsc.subcore_barrier()
        pltpu.sync_copy(acc.at[pl.ds(s * ROWS_PER_TILE, ROWS_PER_TILE)],
                        out_hbm.at[b, pl.ds(s * ROWS_PER_TILE, ROWS_PER_TILE)])

        @pl.when(s == NS - 1)
        def _copy_tail():
            pltpu.sync_copy(acc.at[pl.ds(NS * ROWS_PER_TILE, TAIL_ROWS)],
                            out_hbm.at[b, pl.ds(NS * ROWS_PER_TILE, TAIL_ROWS)])

        plsc.subcore_barrier()


def _sc_scatter(ext, gidx, sidx, zeros_rows):
    mesh = plsc.VectorSubcoreMesh(core_axis_name="c", subcore_axis_name="s")
    return pl.kernel(
        _sc_body,
        out_type=jax.ShapeDtypeStruct((BS, N, D), jnp.float32),
        mesh=mesh,
        scratch_types=[
            pltpu.VMEM((SUP, CH), jnp.int32),
            pltpu.VMEM((SUP, CH), jnp.int32),
            pltpu.VMEM((CH, D), jnp.float32),
            pltpu.VMEM((CH, D), jnp.float32),
            pltpu.VMEM((CH, D), jnp.float32),
            pltpu.VMEM((CH, D), jnp.float32),
            pltpu.VMEM_SHARED((ACC_ROWS, D), jnp.float32),
            pltpu.SemaphoreType.DMA,
            pltpu.SemaphoreType.DMA,
            pltpu.SemaphoreType.DMA,
            pltpu.SemaphoreType.DMA,
        ],
    )(ext, gidx, sidx, zeros_rows)


def kernel(node_embedding, edges, edge_types, emb_table, W0, b0, W1, b1):
    x = node_embedding.reshape(BS * N, D)

    origin = edges[:, :, 0]
    final = edges[:, :, 1]
    npad = NS * PER_TILE - E

    goff = origin + (jnp.arange(BS, dtype=jnp.int32) * N)[:, None]
    pad_g = jnp.arange(npad, dtype=jnp.int32) % 4096
    pad_g = jnp.broadcast_to(pad_g, (BS, npad))
    pad_s = N + (jnp.arange(npad, dtype=jnp.int32) % 16)
    pad_s = jnp.broadcast_to(pad_s, (BS, npad))
    gidx = jnp.concatenate([goff, pad_g], axis=1).reshape(BS, NS, NCHUNK, CH)
    sidx = jnp.concatenate([final, pad_s], axis=1).reshape(BS, NS, NCHUNK, CH)

    pad_h = HDUMP + (jnp.arange(npad, dtype=jnp.int32) % 256)
    pad_h = jnp.broadcast_to(pad_h, (BS, npad))
    hidxp = jnp.concatenate([final * T + edge_types, pad_h], axis=1).reshape(
        BS, NS, NCHUNK_H, CHH)

    zeros_rows = jnp.zeros((ROWS_PER_TILE, D), jnp.float32)
    zerosf = jnp.zeros((HROWS_PER_TILE * T,), jnp.float32)

    hist = _sc_hist(hidxp, zerosf)
    cnt_flat = hist.reshape(BS, NS * HROWS_PER_TILE, T)[:, :N, :].reshape(
        BS * N, T)

    cur = x
    res = None
    tbls = []
    for li, (W, b) in enumerate(((W0, b0), (W1, b1))):
        wg = W[:, :D]
        tbls.append(_type_table(emb_table, W[:, D:], b))
        if li == 0:
            transformed = _transform(cur, wg)
        else:
            cur, transformed = _mid(cur, res, cnt_flat, tbls[0], wg)
        res = _sc_scatter(transformed, gidx, sidx, zeros_rows).reshape(BS * N, D)

    out = _post(cur, res, cnt_flat, tbls[1])
    return out.reshape(BS, N, D)
